# Initial kernel scaffold; baseline (speedup 1.0000x reference)
import jax
import jax.numpy as jnp
from jax.experimental import pallas as pl


def kernel(token_ids, position_ids, segment_ids, token_table, pos_table, seg_table, scale, bias):
    raise NotImplementedError("write your pallas kernel here")


# speedup vs baseline: 2.4797x; 2.4797x over previous
import functools

import jax
import jax.numpy as jnp
from jax import lax
from jax.experimental import pallas as pl
from jax.experimental.pallas import tpu as pltpu
from jax.experimental.pallas import tpu_sc as plsc

H = 128
EPS = 1e-12

_NC = 2
_NS = 16
_NW = _NC * _NS

_CHUNK = 128


def _sc_gather_sum(tok_ids3d, pos_ids3d, token_table, fused_pos_table):
    nw, ch_per_w, chunk = tok_ids3d.shape
    n = nw * ch_per_w * chunk
    per_w = n // _NW

    mesh = plsc.VectorSubcoreMesh(core_axis_name="c", subcore_axis_name="s")

    @functools.partial(
        pl.kernel,
        out_type=jax.ShapeDtypeStruct((n, H), jnp.float32),
        mesh=mesh,
        scratch_types=[
            pltpu.VMEM((ch_per_w, chunk), jnp.int32),
            pltpu.VMEM((ch_per_w, chunk), jnp.int32),
            pltpu.VMEM((chunk, H), jnp.float32),
            pltpu.SemaphoreType.DMA,
        ],
    )
    def k(tok_hbm, pos_hbm, table_hbm, ptab_hbm, out_hbm,
          tok_v, pos_v, rows_v, sem):
        cid = lax.axis_index("c")
        sid = lax.axis_index("s")
        wid = sid * _NC + cid
        row_base = wid * per_w

        pltpu.sync_copy(tok_hbm.at[wid], tok_v)
        pltpu.sync_copy(pos_hbm.at[wid], pos_v)

        def body(j, carry):
            pltpu.async_copy(table_hbm.at[tok_v.at[j]], rows_v, sem).wait()
            pltpu.async_copy(ptab_hbm.at[pos_v.at[j]], rows_v, sem,
                             add=True).wait()
            pltpu.sync_copy(rows_v,
                            out_hbm.at[pl.ds(row_base + j * chunk, chunk)])
            return carry

        lax.fori_loop(0, ch_per_w, body, 0)

    return k(tok_ids3d, pos_ids3d, token_table, fused_pos_table)


def _tc_layernorm(emb, scale, bias):
    m = emb.shape[0]
    blk = 1024

    def body(x_ref, s_ref, b_ref, o_ref):
        x = x_ref[...]
        mu = jnp.mean(x, axis=-1, keepdims=True)
        xc = x - mu
        m2 = jnp.mean(xc * xc, axis=-1, keepdims=True)
        o_ref[...] = xc * lax.rsqrt(m2 + EPS) * s_ref[...] + b_ref[...]

    return pl.pallas_call(
        body,
        grid=(m // blk,),
        in_specs=[
            pl.BlockSpec((blk, H), lambda i: (i, 0)),
            pl.BlockSpec((1, H), lambda i: (0, 0)),
            pl.BlockSpec((1, H), lambda i: (0, 0)),
        ],
        out_specs=pl.BlockSpec((blk, H), lambda i: (i, 0)),
        out_shape=jax.ShapeDtypeStruct((m, H), jnp.float32),
    )(emb, scale.reshape(1, H), bias.reshape(1, H))


def kernel(token_ids, position_ids, segment_ids, token_table, pos_table,
           seg_table, scale, bias):
    b, l = token_ids.shape
    n = b * l
    fused_pos = pos_table + seg_table[0][None, :]

    ch_per_w = n // (_NW * _CHUNK)
    tok3d = token_ids.reshape(_NW, ch_per_w, _CHUNK)
    pos3d = position_ids.reshape(_NW, ch_per_w, _CHUNK)

    emb = _sc_gather_sum(tok3d, pos3d, token_table, fused_pos)
    out = _tc_layernorm(emb, scale, bias)
    return out.reshape(b, l, H)


# speedup vs baseline: 3.2872x; 1.3256x over previous
import functools

import jax
import jax.numpy as jnp
from jax import lax
from jax.experimental import pallas as pl
from jax.experimental.pallas import tpu as pltpu
from jax.experimental.pallas import tpu_sc as plsc

H = 128
EPS = 1e-12

_NC = 2
_NS = 16
_NW = _NC * _NS

_CHUNK = 128


def _sc_gather_sum(tok_ids3d, pos_ids3d, token_table, fused_pos_table):
    nw, ch_per_w, chunk = tok_ids3d.shape
    n = nw * ch_per_w * chunk
    per_w = n // _NW
    assert ch_per_w % 2 == 0
    n_pairs = ch_per_w // 2

    mesh = plsc.VectorSubcoreMesh(core_axis_name="c", subcore_axis_name="s")

    @functools.partial(
        pl.kernel,
        out_type=jax.ShapeDtypeStruct((n, H), jnp.float32),
        mesh=mesh,
        scratch_types=[
            pltpu.VMEM((ch_per_w, chunk), jnp.int32),
            pltpu.VMEM((ch_per_w, chunk), jnp.int32),
            pltpu.VMEM((chunk, H), jnp.float32),
            pltpu.VMEM((chunk, H), jnp.float32),
            pltpu.SemaphoreType.DMA,
            pltpu.SemaphoreType.DMA,
            pltpu.SemaphoreType.DMA,
            pltpu.SemaphoreType.DMA,
            pltpu.SemaphoreType.DMA,
            pltpu.SemaphoreType.DMA,
        ],
    )
    def k(tok_hbm, pos_hbm, table_hbm, ptab_hbm, out_hbm,
          tok_v, pos_v, rows0, rows1,
          sem_g0, sem_g1, sem_a0, sem_a1, sem_s0, sem_s1):
        cid = lax.axis_index("c")
        sid = lax.axis_index("s")
        wid = sid * _NC + cid
        row_base = wid * per_w

        pltpu.sync_copy(tok_hbm.at[wid], tok_v)
        pltpu.sync_copy(pos_hbm.at[wid], pos_v)

        def gather(j, buf, sem):
            return pltpu.async_copy(table_hbm.at[tok_v.at[j]], buf, sem)

        def add_pos(j, buf, sem):
            return pltpu.async_copy(ptab_hbm.at[pos_v.at[j]], buf, sem,
                                    add=True)

        def store(j, buf, sem):
            dst = out_hbm.at[pl.ds(row_base + j * chunk, chunk)]
            return pltpu.async_copy(buf, dst, sem)

        gather(0, rows0, sem_g0)

        def pair_body(i, carry):
            j0 = 2 * i
            j1 = 2 * i + 1
            pltpu.make_async_copy(table_hbm.at[tok_v.at[j0]], rows0,
                                  sem_g0).wait()
            a0 = add_pos(j0, rows0, sem_a0)

            @pl.when(i > 0)
            def _():
                pltpu.make_async_copy(
                    rows1, out_hbm.at[pl.ds(row_base + (j1 - 2) * chunk,
                                            chunk)],
                    sem_s1).wait()

            g1 = gather(j1, rows1, sem_g1)
            a0.wait()
            s0 = store(j0, rows0, sem_s0)
            g1.wait()
            a1 = add_pos(j1, rows1, sem_a1)
            s0.wait()

            @pl.when(i < n_pairs - 1)
            def _():
                gather(j1 + 1, rows0, sem_g0)

            a1.wait()
            store(j1, rows1, sem_s1)
            return carry

        lax.fori_loop(0, n_pairs, pair_body, 0)
        pltpu.make_async_copy(
            rows1, out_hbm.at[pl.ds(row_base + (ch_per_w - 1) * chunk, chunk)],
            sem_s1).wait()

    return k(tok_ids3d, pos_ids3d, token_table, fused_pos_table)


def _tc_layernorm(emb, scale, bias):
    m = emb.shape[0]
    blk = 2048

    def body(x_ref, s_ref, b_ref, o_ref):
        x = x_ref[...]
        mu = jnp.mean(x, axis=-1, keepdims=True)
        xc = x - mu
        m2 = jnp.mean(xc * xc, axis=-1, keepdims=True)
        o_ref[...] = xc * lax.rsqrt(m2 + EPS) * s_ref[...] + b_ref[...]

    return pl.pallas_call(
        body,
        grid=(m // blk,),
        in_specs=[
            pl.BlockSpec((blk, H), lambda i: (i, 0)),
            pl.BlockSpec((1, H), lambda i: (0, 0)),
            pl.BlockSpec((1, H), lambda i: (0, 0)),
        ],
        out_specs=pl.BlockSpec((blk, H), lambda i: (i, 0)),
        out_shape=jax.ShapeDtypeStruct((m, H), jnp.float32),
    )(emb, scale.reshape(1, H), bias.reshape(1, H))


def kernel(token_ids, position_ids, segment_ids, token_table, pos_table,
           seg_table, scale, bias):
    b, l = token_ids.shape
    n = b * l
    fused_pos = pos_table + seg_table[0][None, :]

    ch_per_w = n // (_NW * _CHUNK)
    tok3d = token_ids.reshape(_NW, ch_per_w, _CHUNK)
    pos3d = position_ids.reshape(_NW, ch_per_w, _CHUNK)

    emb = _sc_gather_sum(tok3d, pos3d, token_table, fused_pos)
    out = _tc_layernorm(emb, scale, bias)
    return out.reshape(b, l, H)


# speedup vs baseline: 4.1650x; 1.2670x over previous
import functools

import jax
import jax.numpy as jnp
from jax import lax
from jax.experimental import pallas as pl
from jax.experimental.pallas import tpu as pltpu
from jax.experimental.pallas import tpu_sc as plsc

H = 128
EPS = 1e-12

_NC = 2
_NS = 16
_NW = _NC * _NS

_CHUNK = 128


def _sc_gather_sum(tok_ids3d, pos_ids3d, token_table, fused_pos_table):
    nw, ch_per_w, chunk = tok_ids3d.shape
    n = nw * ch_per_w * chunk
    per_w = n // _NW
    assert ch_per_w % 2 == 0
    n_pairs = ch_per_w // 2

    mesh = plsc.VectorSubcoreMesh(core_axis_name="c", subcore_axis_name="s")

    @functools.partial(
        pl.kernel,
        out_type=jax.ShapeDtypeStruct((n, H), jnp.float32),
        mesh=mesh,
        scratch_types=[
            pltpu.VMEM((ch_per_w, chunk), jnp.int32),
            pltpu.VMEM((ch_per_w, chunk), jnp.int32),
            pltpu.VMEM((chunk, H), jnp.float32),
            pltpu.VMEM((chunk, H), jnp.float32),
            pltpu.VMEM_SHARED((512, H), jnp.float32),
            pltpu.SemaphoreType.DMA,
            pltpu.SemaphoreType.DMA,
            pltpu.SemaphoreType.DMA,
            pltpu.SemaphoreType.DMA,
            pltpu.SemaphoreType.DMA,
            pltpu.SemaphoreType.DMA,
        ],
    )
    def k(tok_hbm, pos_hbm, table_hbm, ptab_hbm, out_hbm,
          tok_v, pos_v, rows0, rows1, ptab_v,
          sem_g0, sem_g1, sem_a0, sem_a1, sem_s0, sem_s1):
        cid = lax.axis_index("c")
        sid = lax.axis_index("s")
        wid = sid * _NC + cid
        row_base = wid * per_w

        pltpu.sync_copy(tok_hbm.at[wid], tok_v)
        pltpu.sync_copy(pos_hbm.at[wid], pos_v)
        @pl.when(sid == 0)
        def _():
            pltpu.sync_copy(ptab_hbm, ptab_v)
        plsc.subcore_barrier()

        def gather(j, buf, sem):
            return pltpu.async_copy(table_hbm.at[tok_v.at[j]], buf, sem)

        def add_pos(j, buf, sem):
            return pltpu.async_copy(ptab_v.at[pos_v.at[j]], buf, sem,
                                    add=True)

        def store(j, buf, sem):
            dst = out_hbm.at[pl.ds(row_base + j * chunk, chunk)]
            return pltpu.async_copy(buf, dst, sem)

        gather(0, rows0, sem_g0)

        def pair_body(i, carry):
            j0 = 2 * i
            j1 = 2 * i + 1
            pltpu.make_async_copy(table_hbm.at[tok_v.at[j0]], rows0,
                                  sem_g0).wait()
            a0 = add_pos(j0, rows0, sem_a0)

            @pl.when(i > 0)
            def _():
                pltpu.make_async_copy(
                    rows1, out_hbm.at[pl.ds(row_base + (j1 - 2) * chunk,
                                            chunk)],
                    sem_s1).wait()

            g1 = gather(j1, rows1, sem_g1)
            a0.wait()
            s0 = store(j0, rows0, sem_s0)
            g1.wait()
            a1 = add_pos(j1, rows1, sem_a1)
            s0.wait()

            @pl.when(i < n_pairs - 1)
            def _():
                gather(j1 + 1, rows0, sem_g0)

            a1.wait()
            store(j1, rows1, sem_s1)
            return carry

        lax.fori_loop(0, n_pairs, pair_body, 0)
        pltpu.make_async_copy(
            rows1, out_hbm.at[pl.ds(row_base + (ch_per_w - 1) * chunk, chunk)],
            sem_s1).wait()

    return k(tok_ids3d, pos_ids3d, token_table, fused_pos_table)


def _tc_layernorm(emb, scale, bias):
    m = emb.shape[0]
    blk = 2048

    def body(x_ref, s_ref, b_ref, o_ref):
        x = x_ref[...]
        mu = jnp.mean(x, axis=-1, keepdims=True)
        xc = x - mu
        m2 = jnp.mean(xc * xc, axis=-1, keepdims=True)
        o_ref[...] = xc * lax.rsqrt(m2 + EPS) * s_ref[...] + b_ref[...]

    return pl.pallas_call(
        body,
        grid=(m // blk,),
        in_specs=[
            pl.BlockSpec((blk, H), lambda i: (i, 0)),
            pl.BlockSpec((1, H), lambda i: (0, 0)),
            pl.BlockSpec((1, H), lambda i: (0, 0)),
        ],
        out_specs=pl.BlockSpec((blk, H), lambda i: (i, 0)),
        out_shape=jax.ShapeDtypeStruct((m, H), jnp.float32),
    )(emb, scale.reshape(1, H), bias.reshape(1, H))


def kernel(token_ids, position_ids, segment_ids, token_table, pos_table,
           seg_table, scale, bias):
    b, l = token_ids.shape
    n = b * l
    fused_pos = pos_table + seg_table[0][None, :]

    ch_per_w = n // (_NW * _CHUNK)
    tok3d = token_ids.reshape(_NW, ch_per_w, _CHUNK)
    pos3d = position_ids.reshape(_NW, ch_per_w, _CHUNK)

    emb = _sc_gather_sum(tok3d, pos3d, token_table, fused_pos)
    out = _tc_layernorm(emb, scale, bias)
    return out.reshape(b, l, H)


# speedup vs baseline: 4.3584x; 1.0464x over previous
import functools

import jax
import jax.numpy as jnp
from jax import lax
from jax.experimental import pallas as pl
from jax.experimental.pallas import tpu as pltpu
from jax.experimental.pallas import tpu_sc as plsc

H = 128
EPS = 1e-12

_NC = 2
_NS = 16
_NW = _NC * _NS

_CHUNK = 128
_NBUF = 4


def _sc_gather_sum(tok_ids3d, pos_ids3d, token_table, fused_pos_table):
    nw, ch_per_w, chunk = tok_ids3d.shape
    n = nw * ch_per_w * chunk
    per_w = n // _NW
    n_pipe = ch_per_w + 2
    p_rows = fused_pos_table.shape[0]

    mesh = plsc.VectorSubcoreMesh(core_axis_name="c", subcore_axis_name="s")

    @functools.partial(
        pl.kernel,
        out_type=jax.ShapeDtypeStruct((n, H), jnp.float32),
        mesh=mesh,
        scratch_types=[
            pltpu.VMEM((ch_per_w, chunk), jnp.int32),
            pltpu.VMEM((ch_per_w, chunk), jnp.int32),
            [pltpu.VMEM((chunk, H), jnp.float32) for _ in range(_NBUF)],
            pltpu.VMEM_SHARED((p_rows, H), jnp.float32),
            [pltpu.SemaphoreType.DMA for _ in range(_NBUF)],
            [pltpu.SemaphoreType.DMA for _ in range(_NBUF)],
            [pltpu.SemaphoreType.DMA for _ in range(_NBUF)],
        ],
    )
    def k(tok_hbm, pos_hbm, table_hbm, ptab_hbm, out_hbm,
          tok_v, pos_v, rows, ptab_s, sem_g, sem_a, sem_s):
        cid = lax.axis_index("c")
        sid = lax.axis_index("s")
        wid = sid * _NC + cid
        row_base = wid * per_w

        pltpu.sync_copy(tok_hbm.at[wid], tok_v)
        pltpu.sync_copy(pos_hbm.at[wid], pos_v)

        @pl.when(sid == 0)
        def _():
            pltpu.sync_copy(ptab_hbm, ptab_s)
        plsc.subcore_barrier()

        def gather(j, b):
            return pltpu.async_copy(table_hbm.at[tok_v.at[j]], rows[b],
                                    sem_g[b])

        def add_pos(j, b):
            return pltpu.async_copy(ptab_s.at[pos_v.at[j]], rows[b],
                                    sem_a[b], add=True)

        def store(j, b):
            dst = out_hbm.at[pl.ds(row_base + j * chunk, chunk)]
            return pltpu.async_copy(rows[b], dst, sem_s[b])

        gather(0, 0)
        gather(1, 1)

        def quad_body(i, carry):
            for b in range(_NBUF):
                j = _NBUF * i + b

                @pl.when(j < ch_per_w)
                def _(j=j, b=b):
                    pltpu.make_async_copy(table_hbm.at[tok_v.at[j]],
                                          rows[b], sem_g[b]).wait()
                    add_pos(j, b)

                jm1 = j - 1
                bm1 = (b - 1) % _NBUF

                @pl.when(jnp.logical_and(jm1 >= 0, jm1 < ch_per_w))
                def _(jm1=jm1, bm1=bm1):
                    pltpu.make_async_copy(ptab_s.at[pos_v.at[jm1]],
                                          rows[bm1], sem_a[bm1]).wait()
                    store(jm1, bm1)

                jm2 = j - 2
                bm2 = (b - 2) % _NBUF

                @pl.when(jnp.logical_and(jm2 >= 0, jm2 < ch_per_w))
                def _(jm2=jm2, bm2=bm2):
                    pltpu.make_async_copy(
                        rows[bm2],
                        out_hbm.at[pl.ds(row_base + jm2 * chunk, chunk)],
                        sem_s[bm2]).wait()

                    @pl.when(jm2 + _NBUF < ch_per_w)
                    def _():
                        gather(jm2 + _NBUF, bm2)

            return carry

        lax.fori_loop(0, (n_pipe + _NBUF - 1) // _NBUF, quad_body, 0)

    return k(tok_ids3d, pos_ids3d, token_table, fused_pos_table)


def _tc_layernorm(emb, scale, bias):
    m = emb.shape[0]
    blk = 2048

    def body(x_ref, s_ref, b_ref, o_ref):
        x = x_ref[...]
        mu = jnp.mean(x, axis=-1, keepdims=True)
        xc = x - mu
        m2 = jnp.mean(xc * xc, axis=-1, keepdims=True)
        o_ref[...] = xc * lax.rsqrt(m2 + EPS) * s_ref[...] + b_ref[...]

    return pl.pallas_call(
        body,
        grid=(m // blk,),
        in_specs=[
            pl.BlockSpec((blk, H), lambda i: (i, 0)),
            pl.BlockSpec((1, H), lambda i: (0, 0)),
            pl.BlockSpec((1, H), lambda i: (0, 0)),
        ],
        out_specs=pl.BlockSpec((blk, H), lambda i: (i, 0)),
        out_shape=jax.ShapeDtypeStruct((m, H), jnp.float32),
    )(emb, scale.reshape(1, H), bias.reshape(1, H))


def kernel(token_ids, position_ids, segment_ids, token_table, pos_table,
           seg_table, scale, bias):
    b, l = token_ids.shape
    n = b * l
    fused_pos = pos_table + seg_table[0][None, :]

    ch_per_w = n // (_NW * _CHUNK)
    tok3d = token_ids.reshape(_NW, ch_per_w, _CHUNK)
    pos3d = position_ids.reshape(_NW, ch_per_w, _CHUNK)

    emb = _sc_gather_sum(tok3d, pos3d, token_table, fused_pos)
    out = _tc_layernorm(emb, scale, bias)
    return out.reshape(b, l, H)
